# Initial kernel scaffold; baseline (speedup 1.0000x reference)
import jax
import jax.numpy as jnp
from jax.experimental import pallas as pl


def kernel(word_ids, table):
    raise NotImplementedError("write your pallas kernel here")


# speedup vs baseline: 1.7966x; 1.7966x over previous
import functools

import jax
import jax.numpy as jnp
from jax import lax
from jax.experimental import pallas as pl
from jax.experimental.pallas import tpu as pltpu
from jax.experimental.pallas import tpu_sc as plsc

_NC = 2
_NS = 16
_NW = _NC * _NS

_CHUNK = 128
_K = 4
_GRP = _CHUNK * _K


@functools.partial(jax.jit, static_argnames=("n_idx", "dim"))
def _sc_gather(table, idx2, *, n_idx, dim):
    per_w = n_idx // _NW
    groups = per_w // _GRP
    chunk_rows_per_w = per_w // _CHUNK

    mesh = plsc.VectorSubcoreMesh(core_axis_name="c", subcore_axis_name="s")

    @functools.partial(
        pl.kernel,
        mesh=mesh,
        compiler_params=pltpu.CompilerParams(use_tc_tiling_on_sc=False),
        out_type=jax.ShapeDtypeStruct((n_idx, dim), jnp.float32),
        scratch_types=[
            pltpu.VMEM((_K, _CHUNK), jnp.int32),
            pltpu.VMEM((_GRP, dim), jnp.float32),
            pltpu.SemaphoreType.DMA,
            pltpu.SemaphoreType.DMA,
        ],
    )
    def k(table_hbm, idx_hbm, out_hbm, idx_v, rows_v, gsem, osem):
        wid = lax.axis_index("s") * _NC + lax.axis_index("c")
        w_chunk_base = wid * chunk_rows_per_w

        def body(g, _):
            chunk_base = w_chunk_base + g * _K
            pltpu.sync_copy(idx_hbm.at[pl.ds(chunk_base, _K)], idx_v)
            cps = []
            for j in range(_K):
                cps.append(
                    pltpu.async_copy(
                        table_hbm.at[idx_v.at[j]],
                        rows_v.at[pl.ds(j * _CHUNK, _CHUNK)],
                        gsem,
                    )
                )
            for cp in cps:
                cp.wait()
            pltpu.sync_copy(
                rows_v, out_hbm.at[pl.ds(chunk_base * _CHUNK, _GRP)]
            )
            return 0

        lax.fori_loop(0, groups, body, 0)

    return k(table, idx2)


def kernel(word_ids, table):
    b, h = word_ids.shape
    v, d = table.shape
    n = b * h
    idx2 = word_ids.reshape(n // _CHUNK, _CHUNK)
    out = _sc_gather(table, idx2, n_idx=n, dim=d)
    return out.reshape(b, h, d)
